# baseline (speedup 1.0000x reference)
import functools

import jax
import jax.numpy as jnp
from jax import lax
from jax.experimental import pallas as pl
from jax.experimental.pallas import tpu as pltpu
from jax.experimental.pallas import tpu_sc as plsc

N_NODES = 10000
N_PAD = 10240
N_EDGES = 320000
EMB = 128
MSG = 64
EPS = 1e-6
NUM_LAYERS = 3

NC, NS = 2, 16
NW = NC * NS
EPW = N_EDGES // NW
CH = 80
NCHUNK = EPW // CH
RPS = N_PAD // NS
NBUF = 5
ROUNDS = NCHUNK // NBUF

_sc_mesh = plsc.VectorSubcoreMesh(
    core_axis_name="c", subcore_axis_name="s", num_cores=NC, num_subcores=NS)
_sc_params = pltpu.CompilerParams(use_tc_tiling_on_sc=False)


@functools.partial(
    pl.kernel,
    out_type=(jax.ShapeDtypeStruct((N_EDGES // 2, MSG), jnp.int32),
              jax.ShapeDtypeStruct((N_EDGES // 2, MSG), jnp.int32)),
    mesh=_sc_mesh,
    compiler_params=_sc_params,
    scratch_types=(
        [pltpu.VMEM((CH,), jnp.int32)] * (2 * NBUF)
        + [pltpu.VMEM((CH, MSG // 2), jnp.int32)] * (2 * NBUF)
        + [pltpu.SemaphoreType.DMA] * (3 * NBUF)
    ),
)
def _sc_gather(sproj, rproj, senders, receivers, g1, g2, *scr):
    sidx = scr[0:NBUF]
    ridx = scr[NBUF:2 * NBUF]
    rows_s = scr[2 * NBUF:3 * NBUF]
    rows_r = scr[3 * NBUF:4 * NBUF]
    sem_idx = scr[4 * NBUF:5 * NBUF]
    sem_gat = scr[5 * NBUF:6 * NBUF]
    sem_wb = scr[6 * NBUF:7 * NBUF]

    wid = lax.axis_index("s") * NC + lax.axis_index("c")
    base0 = wid * EPW

    def start_idx(b, c):
        base = base0 + c * CH
        pltpu.async_copy(senders.at[pl.ds(base, CH)], sidx[b], sem_idx[b])
        pltpu.async_copy(receivers.at[pl.ds(base, CH)], ridx[b], sem_idx[b])

    def wait_idx(b):
        pltpu.make_async_copy(senders.at[pl.ds(0, CH)], sidx[b], sem_idx[b]).wait()
        pltpu.make_async_copy(receivers.at[pl.ds(0, CH)], ridx[b], sem_idx[b]).wait()

    def start_gat(b):
        pltpu.async_copy(sproj.at[sidx[b]], rows_s[b], sem_gat[b])
        pltpu.async_copy(rproj.at[ridx[b]], rows_r[b], sem_gat[b])

    def wait_gat(b):
        pltpu.make_async_copy(sproj.at[sidx[b]], rows_s[b], sem_gat[b]).wait()
        pltpu.make_async_copy(rproj.at[ridx[b]], rows_r[b], sem_gat[b]).wait()

    rhalf = lax.rem(wid, NS) * EPW
    col = jnp.where(wid >= NS, MSG // 2, 0)

    def start_wb(b, c):
        rb = rhalf + c * CH
        pltpu.async_copy(rows_s[b], g1.at[pl.ds(rb, CH), pl.ds(col, MSG // 2)], sem_wb[b])
        pltpu.async_copy(rows_r[b], g2.at[pl.ds(rb, CH), pl.ds(col, MSG // 2)], sem_wb[b])

    def wait_wb(b):
        pltpu.make_async_copy(rows_s[b], g1.at[pl.ds(0, CH), pl.ds(col, MSG // 2)], sem_wb[b]).wait()
        pltpu.make_async_copy(rows_r[b], g2.at[pl.ds(0, CH), pl.ds(col, MSG // 2)], sem_wb[b]).wait()

    for b in range(NBUF):
        start_idx(b, b)

    def round_fn(g, _):
        for b in range(NBUF):
            wait_idx(b)

            @pl.when(g > 0)
            def _():
                wait_wb(b)

            start_gat(b)
        for b in range(NBUF):
            wait_gat(b)
            start_wb(b, g * NBUF + b)
        for b in range(NBUF):
            nxt = jnp.minimum((g + 1) * NBUF + b, NCHUNK - 1)
            start_idx(b, nxt)
        return 0

    lax.fori_loop(0, ROUNDS, round_fn, 0)
    for b in range(NBUF):
        wait_idx(b)
        wait_wb(b)


def _scatter_body(a, receivers, e_norm, out, npart, scr, with_norm):
    ridx = scr[0:NBUF]
    rows = scr[NBUF:2 * NBUF]
    sem_ld = scr[2 * NBUF:3 * NBUF]
    sem_sc = scr[3 * NBUF:4 * NBUF]
    zbuf = scr[4 * NBUF]
    acc = scr[4 * NBUF + 1]
    if with_norm:
        vals = scr[4 * NBUF + 2:5 * NBUF + 2]
        acc_n = scr[5 * NBUF + 2]

    cid = lax.axis_index("c")
    sid = lax.axis_index("s")
    wid = sid * NC + cid

    def zb(i, _):
        zbuf[i // 4, pl.ds((i % 4) * 16, 16)] = jnp.zeros((16,), jnp.float32)
        return 0

    lax.fori_loop(0, 128 * 4, zb, 0)

    def zc(k, _):
        pltpu.sync_copy(zbuf, acc.at[pl.ds(sid * RPS + k * 128, 128)])
        return 0

    lax.fori_loop(0, RPS // 128, zc, 0)
    if with_norm:
        for k in range(RPS // MSG):
            pltpu.sync_copy(zbuf.at[0], acc_n.at[pl.ds(sid * RPS + k * MSG, MSG)])
    plsc.subcore_barrier()

    base0 = wid * EPW
    rhalf = lax.rem(wid, NS) * EPW
    col = jnp.where(wid >= NS, MSG, 0)

    def start_ld(b, c):
        base = base0 + c * CH
        pltpu.async_copy(receivers.at[pl.ds(base, CH)], ridx[b], sem_ld[b])
        pltpu.async_copy(a.at[pl.ds(rhalf + c * CH, CH), pl.ds(col, MSG)],
                         rows[b], sem_ld[b])
        if with_norm:
            pltpu.async_copy(e_norm.at[pl.ds(base, CH)], vals[b], sem_ld[b])

    def wait_ld(b):
        pltpu.make_async_copy(receivers.at[pl.ds(0, CH)], ridx[b], sem_ld[b]).wait()
        pltpu.make_async_copy(a.at[pl.ds(0, CH), pl.ds(col, MSG)],
                              rows[b], sem_ld[b]).wait()
        if with_norm:
            pltpu.make_async_copy(e_norm.at[pl.ds(0, CH)], vals[b], sem_ld[b]).wait()

    for b in range(NBUF):
        start_ld(b, b)

    def round_fn(g, _):
        for b in range(NBUF):
            wait_ld(b)
            pltpu.async_copy(rows[b], acc.at[ridx[b]], sem_sc[b], add=True)
            if with_norm:
                pltpu.async_copy(vals[b], acc_n.at[ridx[b]], sem_sc[b], add=True)
        for b in range(NBUF):
            pltpu.make_async_copy(rows[b], acc.at[ridx[b]], sem_sc[b]).wait()
            if with_norm:
                pltpu.make_async_copy(vals[b], acc_n.at[ridx[b]], sem_sc[b]).wait()
            nxt = jnp.minimum((g + 1) * NBUF + b, NCHUNK - 1)
            start_ld(b, nxt)
        return 0

    lax.fori_loop(0, ROUNDS, round_fn, 0)
    for b in range(NBUF):
        wait_ld(b)
    plsc.subcore_barrier()
    pltpu.sync_copy(acc.at[pl.ds(sid * RPS, RPS)],
                    out.at[cid, pl.ds(sid * RPS, RPS)])
    if with_norm:
        pltpu.sync_copy(acc_n.at[pl.ds(sid * RPS, RPS)],
                        npart.at[cid, pl.ds(sid * RPS, RPS)])


_scatter_scr = (
    [pltpu.VMEM((CH,), jnp.int32)] * NBUF
    + [pltpu.VMEM((CH, MSG), jnp.float32)] * NBUF
    + [pltpu.SemaphoreType.DMA] * (2 * NBUF)
    + [pltpu.VMEM((128, MSG), jnp.float32),
       pltpu.VMEM_SHARED((N_PAD, MSG), jnp.float32)]
)


@functools.partial(
    pl.kernel,
    out_type=(jax.ShapeDtypeStruct((NC, N_PAD, MSG), jnp.float32),
              jax.ShapeDtypeStruct((NC, N_PAD), jnp.float32)),
    mesh=_sc_mesh,
    compiler_params=_sc_params,
    scratch_types=(
        _scatter_scr
        + [pltpu.VMEM((CH,), jnp.float32)] * NBUF
        + [pltpu.VMEM_SHARED((N_PAD,), jnp.float32)]
    ),
)
def _sc_scatter_norm(a, receivers, e_norm, out, npart, *scr):
    _scatter_body(a, receivers, e_norm, out, npart, scr, True)


@functools.partial(
    pl.kernel,
    out_type=jax.ShapeDtypeStruct((NC, N_PAD, MSG), jnp.float32),
    mesh=_sc_mesh,
    compiler_params=_sc_params,
    scratch_types=_scatter_scr,
)
def _sc_scatter(a, receivers, out, *scr):
    _scatter_body(a, receivers, None, out, None, scr, False)


_NB = 2000
_EB = 2000


def _silu(x):
    return x * jax.nn.sigmoid(x)


def _pack16(v):
    n = v.shape[1]
    cols = []
    for g in range(n // MSG):
        half = v[:, g * MSG:(g + 1) * MSG]
        lo = jax.lax.bitcast_convert_type(half[:, :MSG // 2], jnp.int32)
        hi = jax.lax.bitcast_convert_type(half[:, MSG // 2:], jnp.int32)
        lo16 = jax.lax.shift_right_logical(lo + 0x8000, 16)
        hi16 = (hi + 0x8000) & jnp.int32(-65536)
        cols.append(lo16 | hi16)
    return jnp.concatenate(cols, axis=-1) if len(cols) > 1 else cols[0]


def _unpack16(w):
    n = w.shape[1]
    cols = []
    for g in range(n // (MSG // 2)):
        word = w[:, g * (MSG // 2):(g + 1) * (MSG // 2)]
        lo = jax.lax.bitcast_convert_type(
            jax.lax.shift_left(word, 16), jnp.float32)
        hi = jax.lax.bitcast_convert_type(
            word & jnp.int32(-65536), jnp.float32)
        cols.append(jnp.concatenate([lo, hi], axis=-1))
    return jnp.concatenate(cols, axis=-1) if len(cols) > 1 else cols[0]


def _tc_proj0(s_embed, r_embed, mp):
    def body(s_ref, r_ref, w0, b0, w1, b1, w2, b2, wr0,
             sp0, sp1, sp2, rp0):
        s = s_ref[...]

        pack = _pack16
        sp0[...] = pack(jnp.dot(s, w0[...], preferred_element_type=jnp.float32)
                        + b0[...])
        sp1[...] = pack(jnp.dot(s, w1[...], preferred_element_type=jnp.float32)
                        + b1[...])
        sp2[...] = pack(jnp.dot(s, w2[...], preferred_element_type=jnp.float32)
                        + b2[...])
        rp0[...] = pack(jnp.dot(r_ref[...], wr0[...],
                                preferred_element_type=jnp.float32))

    wmat = pl.BlockSpec((EMB, MSG), lambda i: (0, 0))
    wvec = pl.BlockSpec((1, MSG), lambda i: (0, 0))
    nblk = pl.BlockSpec((_NB, MSG // 2), lambda i: (i, 0))
    return pl.pallas_call(
        body,
        grid=(N_NODES // _NB,),
        in_specs=[pl.BlockSpec((_NB, EMB), lambda i: (i, 0)),
                  pl.BlockSpec((_NB, EMB), lambda i: (i, 0)),
                  wmat, wvec, wmat, wvec, wmat, wvec, wmat],
        out_specs=[nblk, nblk, nblk, nblk],
        out_shape=[jax.ShapeDtypeStruct((N_NODES, MSG // 2), jnp.int32)] * 4,
    )(s_embed, r_embed,
      mp[0]["Ws"], mp[0]["bs"].reshape(1, MSG),
      mp[1]["Ws"], mp[1]["bs"].reshape(1, MSG),
      mp[2]["Ws"], mp[2]["bs"].reshape(1, MSG),
      mp[0]["Wr"])


def _tc_pack_edges(e_embed):
    def body(et_ref, eb_ref, out_ref):
        v = jnp.concatenate([et_ref[...], eb_ref[...]], axis=-1)
        out_ref[...] = _pack16(v)

    H = _EB // 2
    NH = N_EDGES // 2 // H
    return pl.pallas_call(
        body,
        grid=(NH,),
        in_specs=[pl.BlockSpec((H, MSG), lambda i: (i, 0)),
                  pl.BlockSpec((H, MSG), lambda i: (i + NH, 0))],
        out_specs=pl.BlockSpec((H, MSG), lambda i: (i, 0)),
        out_shape=jax.ShapeDtypeStruct((N_EDGES // 2, MSG), jnp.int32),
    )(e_embed, e_embed)


def _tc_edge(g1, g2, e_packed, We2, sc2, bi2):
    H = _EB // 2

    def body(g1_ref, g2_ref, e_ref, we_ref, sc_ref, bi_ref, out_ref):
        x = _unpack16(g1_ref[...]) + _unpack16(g2_ref[...])
        xl = x[:, :MSG]
        xr = x[:, MSG:]
        mul = jnp.mean(xl, axis=-1, keepdims=True)
        mur = jnp.mean(xr, axis=-1, keepdims=True)
        xcl = xl - mul
        xcr = xr - mur
        rnl = lax.rsqrt(jnp.mean(xcl * xcl, axis=-1, keepdims=True) + EPS)
        rnr = lax.rsqrt(jnp.mean(xcr * xcr, axis=-1, keepdims=True) + EPS)
        xc = jnp.concatenate([xcl * rnl, xcr * rnr], axis=-1)
        y = _silu(xc * sc_ref[...] + bi_ref[...])
        ew = jnp.dot(_unpack16(e_ref[...]), we_ref[...],
                     preferred_element_type=jnp.float32)
        out_ref[...] = y * ew

    blk = pl.BlockSpec((H, MSG), lambda i: (i, 0))
    obk = pl.BlockSpec((H, 2 * MSG), lambda i: (i, 0))
    return pl.pallas_call(
        body,
        grid=(N_EDGES // _EB,),
        in_specs=[blk, blk, blk,
                  pl.BlockSpec((2 * MSG, 2 * MSG), lambda i: (0, 0)),
                  pl.BlockSpec((1, 2 * MSG), lambda i: (0, 0)),
                  pl.BlockSpec((1, 2 * MSG), lambda i: (0, 0))],
        out_specs=obk,
        out_shape=jax.ShapeDtypeStruct((N_EDGES // 2, 2 * MSG), jnp.float32),
    )(g1, g2, e_packed, We2, sc2, bi2)


def _update_math(p_ref, n_ref, r_ref, wout_ref, uw):
    (w1v, b1v, w1g, b1g, w2v, b2v, w2g, b2g) = uw
    norm = 1.0 / (n_ref[..., 0:1] + n_ref[..., 1:2] + 1.0)
    msum = (p_ref[0] + p_ref[1]) * norm
    msg = _silu(jnp.dot(msum, wout_ref[...], preferred_element_type=jnp.float32))
    r = r_ref[...]
    y = (jnp.dot(r, w1v[...], preferred_element_type=jnp.float32)
         + b1v[...]) * _silu(
        jnp.dot(r, w1g[...], preferred_element_type=jnp.float32)
        + b1g[...]) + msg
    y = (jnp.dot(y, w2v[...], preferred_element_type=jnp.float32)
         + b2v[...]) * _silu(
        jnp.dot(y, w2g[...], preferred_element_type=jnp.float32)
        + b2g[...]) + r
    return y


_wmat = pl.BlockSpec((EMB, EMB), lambda i: (0, 0))
_wvec = pl.BlockSpec((1, EMB), lambda i: (0, 0))
_upd_specs = [
    pl.BlockSpec((NC, _NB, MSG), lambda i: (0, i, 0)),
    pl.BlockSpec((_NB, NC), lambda i: (i, 0)),
    pl.BlockSpec((_NB, EMB), lambda i: (i, 0)),
    pl.BlockSpec((MSG, EMB), lambda i: (0, 0)),
    _wmat, _wvec, _wmat, _wvec, _wmat, _wvec, _wmat, _wvec,
]


def _upd_weights(u):
    return (u["W1v"], u["b1v"].reshape(1, EMB), u["W1g"], u["b1g"].reshape(1, EMB),
            u["W2v"], u["b2v"].reshape(1, EMB), u["W2g"], u["b2g"].reshape(1, EMB))


def _tc_update_proj(partials, npart, r_embed, Wout, u, Wr_next):
    def body(p_ref, n_ref, r_ref, wout_ref,
             w1v, b1v, w1g, b1g, w2v, b2v, w2g, b2g, wrn_ref,
             y_ref, rp_ref):
        y = _update_math(p_ref, n_ref, r_ref, wout_ref,
                         (w1v, b1v, w1g, b1g, w2v, b2v, w2g, b2g))
        y_ref[...] = y
        rp_ref[...] = _pack16(
            jnp.dot(y, wrn_ref[...], preferred_element_type=jnp.float32))

    return pl.pallas_call(
        body,
        grid=(N_NODES // _NB,),
        in_specs=_upd_specs + [pl.BlockSpec((EMB, MSG), lambda i: (0, 0))],
        out_specs=[pl.BlockSpec((_NB, EMB), lambda i: (i, 0)),
                   pl.BlockSpec((_NB, MSG // 2), lambda i: (i, 0))],
        out_shape=[jax.ShapeDtypeStruct((N_NODES, EMB), jnp.float32),
                   jax.ShapeDtypeStruct((N_NODES, MSG // 2), jnp.int32)],
    )(partials, npart, r_embed, Wout, *_upd_weights(u), Wr_next)


def _tc_update_final(partials, npart, r_embed, Wout, u, e1, e2, Wf, bf):
    def body(p_ref, n_ref, r_ref, wout_ref,
             w1v, b1v, w1g, b1g, w2v, b2v, w2g, b2g,
             e1_ref, e2_ref, wf_ref, bf_ref, out_ref):
        y = _update_math(p_ref, n_ref, r_ref, wout_ref,
                         (w1v, b1v, w1g, b1g, w2v, b2v, w2g, b2g))
        w = wf_ref[...]
        acc = jnp.dot(e1_ref[...], w[0:EMB, :], preferred_element_type=jnp.float32)
        acc += jnp.dot(e2_ref[...], w[EMB:2 * EMB, :],
                       preferred_element_type=jnp.float32)
        acc += jnp.dot(y, w[2 * EMB:3 * EMB, :], preferred_element_type=jnp.float32)
        out_ref[...] = acc + bf_ref[...]

    return pl.pallas_call(
        body,
        grid=(N_NODES // _NB,),
        in_specs=_upd_specs + [
            pl.BlockSpec((_NB, EMB), lambda i: (i, 0)),
            pl.BlockSpec((_NB, EMB), lambda i: (i, 0)),
            pl.BlockSpec((NUM_LAYERS * EMB, EMB), lambda i: (0, 0)),
            pl.BlockSpec((1, EMB), lambda i: (0, 0)),
        ],
        out_specs=pl.BlockSpec((_NB, EMB), lambda i: (i, 0)),
        out_shape=jax.ShapeDtypeStruct((N_NODES, EMB), jnp.float32),
    )(partials, npart, r_embed, Wout, *_upd_weights(u), e1, e2, Wf,
      bf.reshape(1, EMB))


def kernel(s_embed, r_embed, e_embed, e_norm, params, senders, receivers):
    senders = senders.astype(jnp.int32)
    receivers = receivers.astype(jnp.int32)
    mp = params["mp"]
    up = params["up"]

    sp0, sp1, sp2, rp = _tc_proj0(s_embed, r_embed, mp)
    sps = (sp0, sp1, sp2)
    e_packed = _tc_pack_edges(e_embed)

    npart = None
    embs = []
    for l in range(NUM_LAYERS):
        We = mp[l]["We"]
        z = jnp.zeros_like(We)
        We2 = jnp.block([[We, z], [z, We]])
        sc2 = jnp.tile(mp[l]["ln_scale"], 2).reshape(1, 2 * MSG)
        bi2 = jnp.tile(mp[l]["ln_bias"], 2).reshape(1, 2 * MSG)
        g1, g2 = _sc_gather(sps[l], rp, senders, receivers)
        a = _tc_edge(g1, g2, e_packed, We2, sc2, bi2)
        if l == 0:
            part, npart = _sc_scatter_norm(a, receivers, e_norm)
            npart = npart.T
        else:
            part = _sc_scatter(a, receivers)
        if l < NUM_LAYERS - 1:
            r_embed, rp = _tc_update_proj(part, npart, r_embed, mp[l]["Wout"],
                                          up[l], mp[l + 1]["Wr"])
            embs.append(r_embed)
        else:
            out = _tc_update_final(part, npart, r_embed, mp[l]["Wout"], up[l],
                                   embs[0], embs[1], params["Wf"], params["bf"])
    return out


# speedup vs baseline: 1.4709x; 1.4709x over previous
import functools

import jax
import jax.numpy as jnp
from jax import lax
from jax.experimental import pallas as pl
from jax.experimental.pallas import tpu as pltpu
from jax.experimental.pallas import tpu_sc as plsc

N_NODES = 10000
N_PAD = 10240
N_EDGES = 320000
EMB = 128
MSG = 64
EPS = 1e-6
NUM_LAYERS = 3

NC, NS = 2, 16
NW = NC * NS
EPW = N_EDGES // NW
CH = 80
NCHUNK = EPW // CH
RPS = N_PAD // NS
NBUF = 5
ROUNDS = NCHUNK // NBUF

_sc_mesh = plsc.VectorSubcoreMesh(
    core_axis_name="c", subcore_axis_name="s", num_cores=NC, num_subcores=NS)
_sc_params = pltpu.CompilerParams(use_tc_tiling_on_sc=False)


@functools.partial(
    pl.kernel,
    out_type=(jax.ShapeDtypeStruct((N_EDGES // 2, 2 * MSG), jnp.float32),
              jax.ShapeDtypeStruct((N_EDGES // 2, 2 * MSG), jnp.float32)),
    mesh=_sc_mesh,
    compiler_params=_sc_params,
    scratch_types=(
        [pltpu.VMEM((CH,), jnp.int32)] * (2 * NBUF)
        + [pltpu.VMEM((CH, MSG), jnp.float32)] * (2 * NBUF)
        + [pltpu.SemaphoreType.DMA] * (3 * NBUF)
    ),
)
def _sc_gather(sproj, rproj, senders, receivers, g1, g2, *scr):
    sidx = scr[0:NBUF]
    ridx = scr[NBUF:2 * NBUF]
    rows_s = scr[2 * NBUF:3 * NBUF]
    rows_r = scr[3 * NBUF:4 * NBUF]
    sem_idx = scr[4 * NBUF:5 * NBUF]
    sem_gat = scr[5 * NBUF:6 * NBUF]
    sem_wb = scr[6 * NBUF:7 * NBUF]

    wid = lax.axis_index("s") * NC + lax.axis_index("c")
    base0 = wid * EPW

    def start_idx(b, c):
        base = base0 + c * CH
        pltpu.async_copy(senders.at[pl.ds(base, CH)], sidx[b], sem_idx[b])
        pltpu.async_copy(receivers.at[pl.ds(base, CH)], ridx[b], sem_idx[b])

    def wait_idx(b):
        pltpu.make_async_copy(senders.at[pl.ds(0, CH)], sidx[b], sem_idx[b]).wait()
        pltpu.make_async_copy(receivers.at[pl.ds(0, CH)], ridx[b], sem_idx[b]).wait()

    def start_gat(b):
        pltpu.async_copy(sproj.at[sidx[b]], rows_s[b], sem_gat[b])
        pltpu.async_copy(rproj.at[ridx[b]], rows_r[b], sem_gat[b])

    def wait_gat(b):
        pltpu.make_async_copy(sproj.at[sidx[b]], rows_s[b], sem_gat[b]).wait()
        pltpu.make_async_copy(rproj.at[ridx[b]], rows_r[b], sem_gat[b]).wait()

    rhalf = lax.rem(wid, NS) * EPW
    col = jnp.where(wid >= NS, MSG, 0)

    def start_wb(b, c):
        rb = rhalf + c * CH
        pltpu.async_copy(rows_s[b], g1.at[pl.ds(rb, CH), pl.ds(col, MSG)], sem_wb[b])
        pltpu.async_copy(rows_r[b], g2.at[pl.ds(rb, CH), pl.ds(col, MSG)], sem_wb[b])

    def wait_wb(b):
        pltpu.make_async_copy(rows_s[b], g1.at[pl.ds(0, CH), pl.ds(col, MSG)], sem_wb[b]).wait()
        pltpu.make_async_copy(rows_r[b], g2.at[pl.ds(0, CH), pl.ds(col, MSG)], sem_wb[b]).wait()

    for b in range(NBUF):
        start_idx(b, b)

    def round_fn(g, _):
        for b in range(NBUF):
            wait_idx(b)

            @pl.when(g > 0)
            def _():
                wait_wb(b)

            start_gat(b)
        for b in range(NBUF):
            wait_gat(b)
            start_wb(b, g * NBUF + b)
        for b in range(NBUF):
            nxt = jnp.minimum((g + 1) * NBUF + b, NCHUNK - 1)
            start_idx(b, nxt)
        return 0

    lax.fori_loop(0, ROUNDS, round_fn, 0)
    for b in range(NBUF):
        wait_idx(b)
        wait_wb(b)


def _scatter_body(a, receivers, e_norm, out, npart, scr, with_norm):
    ridx = scr[0:NBUF]
    rows = scr[NBUF:2 * NBUF]
    sem_ld = scr[2 * NBUF:3 * NBUF]
    sem_sc = scr[3 * NBUF:4 * NBUF]
    zbuf = scr[4 * NBUF]
    acc = scr[4 * NBUF + 1]
    if with_norm:
        vals = scr[4 * NBUF + 2:5 * NBUF + 2]
        acc_n = scr[5 * NBUF + 2]

    cid = lax.axis_index("c")
    sid = lax.axis_index("s")
    wid = sid * NC + cid

    def zb(i, _):
        zbuf[i // 4, pl.ds((i % 4) * 16, 16)] = jnp.zeros((16,), jnp.float32)
        return 0

    lax.fori_loop(0, 128 * 4, zb, 0)

    def zc(k, _):
        pltpu.sync_copy(zbuf, acc.at[pl.ds(sid * RPS + k * 128, 128)])
        return 0

    lax.fori_loop(0, RPS // 128, zc, 0)
    if with_norm:
        for k in range(RPS // MSG):
            pltpu.sync_copy(zbuf.at[0], acc_n.at[pl.ds(sid * RPS + k * MSG, MSG)])
    plsc.subcore_barrier()

    base0 = wid * EPW
    rhalf = lax.rem(wid, NS) * EPW
    col = jnp.where(wid >= NS, MSG, 0)

    def start_ld(b, c):
        base = base0 + c * CH
        pltpu.async_copy(receivers.at[pl.ds(base, CH)], ridx[b], sem_ld[b])
        pltpu.async_copy(a.at[pl.ds(rhalf + c * CH, CH), pl.ds(col, MSG)],
                         rows[b], sem_ld[b])
        if with_norm:
            pltpu.async_copy(e_norm.at[pl.ds(base, CH)], vals[b], sem_ld[b])

    def wait_ld(b):
        pltpu.make_async_copy(receivers.at[pl.ds(0, CH)], ridx[b], sem_ld[b]).wait()
        pltpu.make_async_copy(a.at[pl.ds(0, CH), pl.ds(col, MSG)],
                              rows[b], sem_ld[b]).wait()
        if with_norm:
            pltpu.make_async_copy(e_norm.at[pl.ds(0, CH)], vals[b], sem_ld[b]).wait()

    for b in range(NBUF):
        start_ld(b, b)

    def round_fn(g, _):
        for b in range(NBUF):
            wait_ld(b)
            pltpu.async_copy(rows[b], acc.at[ridx[b]], sem_sc[b], add=True)
            if with_norm:
                pltpu.async_copy(vals[b], acc_n.at[ridx[b]], sem_sc[b], add=True)
        for b in range(NBUF):
            pltpu.make_async_copy(rows[b], acc.at[ridx[b]], sem_sc[b]).wait()
            if with_norm:
                pltpu.make_async_copy(vals[b], acc_n.at[ridx[b]], sem_sc[b]).wait()
            nxt = jnp.minimum((g + 1) * NBUF + b, NCHUNK - 1)
            start_ld(b, nxt)
        return 0

    lax.fori_loop(0, ROUNDS, round_fn, 0)
    for b in range(NBUF):
        wait_ld(b)
    plsc.subcore_barrier()
    pltpu.sync_copy(acc.at[pl.ds(sid * RPS, RPS)],
                    out.at[cid, pl.ds(sid * RPS, RPS)])
    if with_norm:
        pltpu.sync_copy(acc_n.at[pl.ds(sid * RPS, RPS)],
                        npart.at[cid, pl.ds(sid * RPS, RPS)])


_scatter_scr = (
    [pltpu.VMEM((CH,), jnp.int32)] * NBUF
    + [pltpu.VMEM((CH, MSG), jnp.float32)] * NBUF
    + [pltpu.SemaphoreType.DMA] * (2 * NBUF)
    + [pltpu.VMEM((128, MSG), jnp.float32),
       pltpu.VMEM_SHARED((N_PAD, MSG), jnp.float32)]
)


@functools.partial(
    pl.kernel,
    out_type=(jax.ShapeDtypeStruct((NC, N_PAD, MSG), jnp.float32),
              jax.ShapeDtypeStruct((NC, N_PAD), jnp.float32)),
    mesh=_sc_mesh,
    compiler_params=_sc_params,
    scratch_types=(
        _scatter_scr
        + [pltpu.VMEM((CH,), jnp.float32)] * NBUF
        + [pltpu.VMEM_SHARED((N_PAD,), jnp.float32)]
    ),
)
def _sc_scatter_norm(a, receivers, e_norm, out, npart, *scr):
    _scatter_body(a, receivers, e_norm, out, npart, scr, True)


@functools.partial(
    pl.kernel,
    out_type=jax.ShapeDtypeStruct((NC, N_PAD, MSG), jnp.float32),
    mesh=_sc_mesh,
    compiler_params=_sc_params,
    scratch_types=_scatter_scr,
)
def _sc_scatter(a, receivers, out, *scr):
    _scatter_body(a, receivers, None, out, None, scr, False)


_NB = 2000
_EB = 2000


def _silu(x):
    return x * jax.nn.sigmoid(x)


def _tc_proj0(s_embed, r_embed, mp):
    def body(s_ref, r_ref, w0, b0, w1, b1, w2, b2, wr0,
             sp0, sp1, sp2, rp0):
        s = s_ref[...]
        sp0[...] = jnp.dot(s, w0[...], preferred_element_type=jnp.float32) + b0[...]
        sp1[...] = jnp.dot(s, w1[...], preferred_element_type=jnp.float32) + b1[...]
        sp2[...] = jnp.dot(s, w2[...], preferred_element_type=jnp.float32) + b2[...]
        rp0[...] = jnp.dot(r_ref[...], wr0[...], preferred_element_type=jnp.float32)

    wmat = pl.BlockSpec((EMB, MSG), lambda i: (0, 0))
    wvec = pl.BlockSpec((1, MSG), lambda i: (0, 0))
    nblk = pl.BlockSpec((_NB, MSG), lambda i: (i, 0))
    return pl.pallas_call(
        body,
        grid=(N_NODES // _NB,),
        in_specs=[pl.BlockSpec((_NB, EMB), lambda i: (i, 0)),
                  pl.BlockSpec((_NB, EMB), lambda i: (i, 0)),
                  wmat, wvec, wmat, wvec, wmat, wvec, wmat],
        out_specs=[nblk, nblk, nblk, nblk],
        out_shape=[jax.ShapeDtypeStruct((N_NODES, MSG), jnp.float32)] * 4,
    )(s_embed, r_embed,
      mp[0]["Ws"], mp[0]["bs"].reshape(1, MSG),
      mp[1]["Ws"], mp[1]["bs"].reshape(1, MSG),
      mp[2]["Ws"], mp[2]["bs"].reshape(1, MSG),
      mp[0]["Wr"])


def _tc_pack_edges(et):
    def body(et_ref, eb_ref, out_ref):
        out_ref[...] = jnp.concatenate(
            [et_ref[...].T, eb_ref[...].T], axis=-1)

    H = 1280
    NH = N_EDGES // 2 // H
    return pl.pallas_call(
        body,
        grid=(NH,),
        in_specs=[pl.BlockSpec((MSG, H), lambda i: (0, i)),
                  pl.BlockSpec((MSG, H), lambda i: (0, i + NH))],
        out_specs=pl.BlockSpec((H, 2 * MSG), lambda i: (i, 0)),
        out_shape=jax.ShapeDtypeStruct((N_EDGES // 2, 2 * MSG), jnp.float32),
    )(et, et)


def _tc_edge(g1, g2, e_packed, We2, sc2, bi2):
    H = _EB // 2

    def body(g1_ref, g2_ref, e_ref, we_ref, sc_ref, bi_ref, out_ref):
        x = g1_ref[...] + g2_ref[...]
        xl = x[:, :MSG]
        xr = x[:, MSG:]
        mul = jnp.mean(xl, axis=-1, keepdims=True)
        mur = jnp.mean(xr, axis=-1, keepdims=True)
        xcl = xl - mul
        xcr = xr - mur
        rnl = lax.rsqrt(jnp.mean(xcl * xcl, axis=-1, keepdims=True) + EPS)
        rnr = lax.rsqrt(jnp.mean(xcr * xcr, axis=-1, keepdims=True) + EPS)
        xc = jnp.concatenate([xcl * rnl, xcr * rnr], axis=-1)
        y = _silu(xc * sc_ref[...] + bi_ref[...])
        ew = jnp.dot(e_ref[...], we_ref[...], preferred_element_type=jnp.float32)
        out_ref[...] = y * ew

    blk = pl.BlockSpec((H, 2 * MSG), lambda i: (i, 0))
    return pl.pallas_call(
        body,
        grid=(N_EDGES // _EB,),
        in_specs=[blk, blk, blk,
                  pl.BlockSpec((2 * MSG, 2 * MSG), lambda i: (0, 0)),
                  pl.BlockSpec((1, 2 * MSG), lambda i: (0, 0)),
                  pl.BlockSpec((1, 2 * MSG), lambda i: (0, 0))],
        out_specs=blk,
        out_shape=jax.ShapeDtypeStruct((N_EDGES // 2, 2 * MSG), jnp.float32),
    )(g1, g2, e_packed, We2, sc2, bi2)


def _update_math(p_ref, n_ref, r_ref, wout_ref, uw):
    (w1v, b1v, w1g, b1g, w2v, b2v, w2g, b2g) = uw
    norm = 1.0 / (n_ref[..., 0:1] + n_ref[..., 1:2] + 1.0)
    msum = (p_ref[0] + p_ref[1]) * norm
    msg = _silu(jnp.dot(msum, wout_ref[...], preferred_element_type=jnp.float32))
    r = r_ref[...]
    y = (jnp.dot(r, w1v[...], preferred_element_type=jnp.float32)
         + b1v[...]) * _silu(
        jnp.dot(r, w1g[...], preferred_element_type=jnp.float32)
        + b1g[...]) + msg
    y = (jnp.dot(y, w2v[...], preferred_element_type=jnp.float32)
         + b2v[...]) * _silu(
        jnp.dot(y, w2g[...], preferred_element_type=jnp.float32)
        + b2g[...]) + r
    return y


_wmat = pl.BlockSpec((EMB, EMB), lambda i: (0, 0))
_wvec = pl.BlockSpec((1, EMB), lambda i: (0, 0))
_upd_specs = [
    pl.BlockSpec((NC, _NB, MSG), lambda i: (0, i, 0)),
    pl.BlockSpec((_NB, NC), lambda i: (i, 0)),
    pl.BlockSpec((_NB, EMB), lambda i: (i, 0)),
    pl.BlockSpec((MSG, EMB), lambda i: (0, 0)),
    _wmat, _wvec, _wmat, _wvec, _wmat, _wvec, _wmat, _wvec,
]


def _upd_weights(u):
    return (u["W1v"], u["b1v"].reshape(1, EMB), u["W1g"], u["b1g"].reshape(1, EMB),
            u["W2v"], u["b2v"].reshape(1, EMB), u["W2g"], u["b2g"].reshape(1, EMB))


def _tc_update_proj(partials, npart, r_embed, Wout, u, Wr_next):
    def body(p_ref, n_ref, r_ref, wout_ref,
             w1v, b1v, w1g, b1g, w2v, b2v, w2g, b2g, wrn_ref,
             y_ref, rp_ref):
        y = _update_math(p_ref, n_ref, r_ref, wout_ref,
                         (w1v, b1v, w1g, b1g, w2v, b2v, w2g, b2g))
        y_ref[...] = y
        rp_ref[...] = jnp.dot(y, wrn_ref[...], preferred_element_type=jnp.float32)

    return pl.pallas_call(
        body,
        grid=(N_NODES // _NB,),
        in_specs=_upd_specs + [pl.BlockSpec((EMB, MSG), lambda i: (0, 0))],
        out_specs=[pl.BlockSpec((_NB, EMB), lambda i: (i, 0)),
                   pl.BlockSpec((_NB, MSG), lambda i: (i, 0))],
        out_shape=[jax.ShapeDtypeStruct((N_NODES, EMB), jnp.float32),
                   jax.ShapeDtypeStruct((N_NODES, MSG), jnp.float32)],
    )(partials, npart, r_embed, Wout, *_upd_weights(u), Wr_next)


def _tc_update_final(partials, npart, r_embed, Wout, u, e1, e2, Wf, bf):
    def body(p_ref, n_ref, r_ref, wout_ref,
             w1v, b1v, w1g, b1g, w2v, b2v, w2g, b2g,
             e1_ref, e2_ref, wf_ref, bf_ref, out_ref):
        y = _update_math(p_ref, n_ref, r_ref, wout_ref,
                         (w1v, b1v, w1g, b1g, w2v, b2v, w2g, b2g))
        w = wf_ref[...]
        acc = jnp.dot(e1_ref[...], w[0:EMB, :], preferred_element_type=jnp.float32)
        acc += jnp.dot(e2_ref[...], w[EMB:2 * EMB, :],
                       preferred_element_type=jnp.float32)
        acc += jnp.dot(y, w[2 * EMB:3 * EMB, :], preferred_element_type=jnp.float32)
        out_ref[...] = acc + bf_ref[...]

    return pl.pallas_call(
        body,
        grid=(N_NODES // _NB,),
        in_specs=_upd_specs + [
            pl.BlockSpec((_NB, EMB), lambda i: (i, 0)),
            pl.BlockSpec((_NB, EMB), lambda i: (i, 0)),
            pl.BlockSpec((NUM_LAYERS * EMB, EMB), lambda i: (0, 0)),
            pl.BlockSpec((1, EMB), lambda i: (0, 0)),
        ],
        out_specs=pl.BlockSpec((_NB, EMB), lambda i: (i, 0)),
        out_shape=jax.ShapeDtypeStruct((N_NODES, EMB), jnp.float32),
    )(partials, npart, r_embed, Wout, *_upd_weights(u), e1, e2, Wf,
      bf.reshape(1, EMB))


def kernel(s_embed, r_embed, e_embed, e_norm, params, senders, receivers):
    senders = senders.astype(jnp.int32)
    receivers = receivers.astype(jnp.int32)
    mp = params["mp"]
    up = params["up"]

    sp0, sp1, sp2, rp = _tc_proj0(s_embed, r_embed, mp)
    sps = (sp0, sp1, sp2)
    e_packed = _tc_pack_edges(e_embed.T)

    npart = None
    embs = []
    for l in range(NUM_LAYERS):
        We = mp[l]["We"]
        z = jnp.zeros_like(We)
        We2 = jnp.block([[We, z], [z, We]])
        sc2 = jnp.tile(mp[l]["ln_scale"], 2).reshape(1, 2 * MSG)
        bi2 = jnp.tile(mp[l]["ln_bias"], 2).reshape(1, 2 * MSG)
        g1, g2 = _sc_gather(sps[l], rp, senders, receivers)
        a = _tc_edge(g1, g2, e_packed, We2, sc2, bi2)
        if l == 0:
            part, npart = _sc_scatter_norm(a, receivers, e_norm)
            npart = npart.T
        else:
            part = _sc_scatter(a, receivers)
        if l < NUM_LAYERS - 1:
            r_embed, rp = _tc_update_proj(part, npart, r_embed, mp[l]["Wout"],
                                          up[l], mp[l + 1]["Wr"])
            embs.append(r_embed)
        else:
            out = _tc_update_final(part, npart, r_embed, mp[l]["Wout"], up[l],
                                   embs[0], embs[1], params["Wf"], params["bf"])
    return out
